# baseline (speedup 1.0000x reference)
import jax
import jax.numpy as jnp
from jax.experimental import pallas as pl
from jax.experimental.pallas import tpu as pltpu

_B = 16
_C = 3
_P = 384 * 384
_BLK = 2304
_EPS = 1e-8


def _body(o_ref, t_ref, out_ref, acc_ref, cnt_ref):
    i = pl.program_id(0)

    @pl.when(i == 0)
    def _init():
        acc_ref[...] = jnp.zeros_like(acc_ref)
        cnt_ref[...] = jnp.zeros_like(cnt_ref)

    o = o_ref[...]
    t = t_ref[...]
    dot = jnp.sum(o * t, axis=1)
    no2 = jnp.sum(o * o, axis=1)
    nt2 = jnp.sum(t * t, axis=1)
    norm_o = jnp.sqrt(no2)
    norm_t = jnp.sqrt(nt2)
    mask = (norm_t != 0).astype(jnp.float32)
    cos = dot / (jnp.maximum(norm_o, _EPS) * jnp.maximum(norm_t, _EPS))
    acc_ref[...] += mask * (1.0 - cos)
    cnt_ref[...] += mask

    @pl.when(i == pl.num_programs(0) - 1)
    def _fin():
        loss = jnp.sum(acc_ref[...]) / jnp.sum(cnt_ref[...])
        out_ref[...] = loss.reshape(1, 1)


def kernel(output, target):
    o = output.reshape(_B, _C, _P)
    t = target.reshape(_B, _C, _P)
    grid = (_P // _BLK,)
    out = pl.pallas_call(
        _body,
        grid=grid,
        in_specs=[
            pl.BlockSpec((_B, _C, _BLK), lambda i: (0, 0, i)),
            pl.BlockSpec((_B, _C, _BLK), lambda i: (0, 0, i)),
        ],
        out_specs=pl.BlockSpec((1, 1), lambda i: (0, 0)),
        out_shape=jax.ShapeDtypeStruct((1, 1), jnp.float32),
        scratch_shapes=[
            pltpu.VMEM((_B, _BLK), jnp.float32),
            pltpu.VMEM((_B, _BLK), jnp.float32),
        ],
    )(o, t)
    return out[0, 0]


# speedup vs baseline: 3.0270x; 3.0270x over previous
import jax
import jax.numpy as jnp
from jax.experimental import pallas as pl
from jax.experimental.pallas import tpu as pltpu

_B = 16
_C = 3
_ROWS = 1152
_LANES = 128
_R = 72
_EPS = 1e-8


def _body(o_ref, t_ref, out_ref, acc_ref, cnt_ref):
    i = pl.program_id(0)

    @pl.when(i == 0)
    def _init():
        acc_ref[...] = jnp.zeros_like(acc_ref)
        cnt_ref[...] = jnp.zeros_like(cnt_ref)

    o = o_ref[...]
    t = t_ref[...]
    dot = jnp.sum(o * t, axis=1)
    no2 = jnp.sum(o * o, axis=1)
    nt2 = jnp.sum(t * t, axis=1)
    norm_o = jnp.sqrt(no2)
    norm_t = jnp.sqrt(nt2)
    mask = (norm_t != 0).astype(jnp.float32)
    denom = jnp.maximum(norm_o, _EPS) * jnp.maximum(norm_t, _EPS)
    cos = dot / denom
    acc_ref[...] += mask * (1.0 - cos)
    cnt_ref[...] += mask

    @pl.when(i == pl.num_programs(0) - 1)
    def _fin():
        loss = jnp.sum(acc_ref[...]) / jnp.sum(cnt_ref[...])
        out_ref[...] = loss.reshape(1, 1)


def kernel(output, target):
    o = output.reshape(_B, _C, _ROWS, _LANES)
    t = target.reshape(_B, _C, _ROWS, _LANES)
    grid = (_ROWS // _R,)
    out = pl.pallas_call(
        _body,
        grid=grid,
        in_specs=[
            pl.BlockSpec((_B, _C, _R, _LANES), lambda i: (0, 0, i, 0)),
            pl.BlockSpec((_B, _C, _R, _LANES), lambda i: (0, 0, i, 0)),
        ],
        out_specs=pl.BlockSpec((1, 1), lambda i: (0, 0)),
        out_shape=jax.ShapeDtypeStruct((1, 1), jnp.float32),
        scratch_shapes=[
            pltpu.VMEM((_B, _R, _LANES), jnp.float32),
            pltpu.VMEM((_B, _R, _LANES), jnp.float32),
        ],
    )(o, t)
    return out[0, 0]


# speedup vs baseline: 3.1257x; 1.0326x over previous
import jax
import jax.numpy as jnp
from jax import lax
from jax.experimental import pallas as pl
from jax.experimental.pallas import tpu as pltpu

_B = 16
_C = 3
_ROWS = 1152
_LANES = 128
_EPS2 = 1e-16


def _body(o_ref, t_ref, out_ref, acc_ref, cnt_ref):
    i = pl.program_id(0)

    @pl.when(i == 0)
    def _init():
        acc_ref[...] = jnp.zeros_like(acc_ref)
        cnt_ref[...] = jnp.zeros_like(cnt_ref)

    o = o_ref[0]
    t = t_ref[0]
    dot = jnp.sum(o * t, axis=0)
    no2 = jnp.sum(o * o, axis=0)
    nt2 = jnp.sum(t * t, axis=0)
    r = lax.rsqrt(jnp.maximum(no2, _EPS2) * jnp.maximum(nt2, _EPS2))
    mask = nt2 > 0.0
    acc_ref[...] += jnp.where(mask, 1.0 - dot * r, 0.0)
    cnt_ref[...] += jnp.where(mask, 1.0, 0.0)

    @pl.when(i == pl.num_programs(0) - 1)
    def _fin():
        loss = jnp.sum(acc_ref[...]) / jnp.sum(cnt_ref[...])
        out_ref[...] = loss.reshape(1, 1)


def kernel(output, target):
    o = output.reshape(_B, _C, _ROWS, _LANES)
    t = target.reshape(_B, _C, _ROWS, _LANES)
    out = pl.pallas_call(
        _body,
        grid=(_B,),
        in_specs=[
            pl.BlockSpec((1, _C, _ROWS, _LANES), lambda i: (i, 0, 0, 0)),
            pl.BlockSpec((1, _C, _ROWS, _LANES), lambda i: (i, 0, 0, 0)),
        ],
        out_specs=pl.BlockSpec((1, 1), lambda i: (0, 0)),
        out_shape=jax.ShapeDtypeStruct((1, 1), jnp.float32),
        scratch_shapes=[
            pltpu.VMEM((_ROWS, _LANES), jnp.float32),
            pltpu.VMEM((_ROWS, _LANES), jnp.float32),
        ],
    )(o, t)
    return out[0, 0]


# speedup vs baseline: 11.2596x; 3.6023x over previous
import jax
import jax.numpy as jnp
from jax import lax
from jax.experimental import pallas as pl
from jax.experimental.pallas import tpu as pltpu

_B = 16
_C = 3
_H = 384
_W = 384
_BB = 2
_EPS2 = 1e-16


def _body(o_ref, t_ref, out_ref, acc_ref, cnt_ref):
    i = pl.program_id(0)

    @pl.when(i == 0)
    def _init():
        acc_ref[...] = jnp.zeros_like(acc_ref)
        cnt_ref[...] = jnp.zeros_like(cnt_ref)

    o = o_ref[...]
    t = t_ref[...]
    dot = jnp.sum(o * t, axis=1)
    no2 = jnp.sum(o * o, axis=1)
    nt2 = jnp.sum(t * t, axis=1)
    r = lax.rsqrt(jnp.maximum(no2, _EPS2) * jnp.maximum(nt2, _EPS2))
    mask = nt2 > 0.0
    contrib = jnp.where(mask, 1.0 - dot * r, 0.0)
    cnt_v = jnp.where(mask, 1.0, 0.0)
    acc_ref[...] += jnp.sum(contrib.reshape(_BB * _H // 8, 8, _W), axis=0)
    cnt_ref[...] += jnp.sum(cnt_v.reshape(_BB * _H // 8, 8, _W), axis=0)

    @pl.when(i == pl.num_programs(0) - 1)
    def _fin():
        loss = jnp.sum(acc_ref[...]) / jnp.sum(cnt_ref[...])
        out_ref[...] = loss.reshape(1, 1)


def kernel(output, target):
    out = pl.pallas_call(
        _body,
        grid=(_B // _BB,),
        in_specs=[
            pl.BlockSpec((_BB, _C, _H, _W), lambda i: (i, 0, 0, 0)),
            pl.BlockSpec((_BB, _C, _H, _W), lambda i: (i, 0, 0, 0)),
        ],
        out_specs=pl.BlockSpec((1, 1), lambda i: (0, 0)),
        out_shape=jax.ShapeDtypeStruct((1, 1), jnp.float32),
        scratch_shapes=[
            pltpu.VMEM((8, _W), jnp.float32),
            pltpu.VMEM((8, _W), jnp.float32),
        ],
    )(output, target)
    return out[0, 0]
